# baseline (speedup 1.0000x reference)
import functools

import jax
import jax.numpy as jnp
from jax import lax
from jax.experimental import pallas as pl
from jax.experimental.pallas import tpu as pltpu

_NCHUNK = 4


def _gemm_kernel(x_ref, w_hbm, o_ref, wb_ref, wf_ref, sems):
    K = wb_ref.shape[0]
    ck = K // _NCHUNK
    x_bf = x_ref[...].astype(jnp.bfloat16)

    @pl.when(pl.program_id(1) == 0)
    def _stream_weight_and_compute():
        for c in range(_NCHUNK):
            pltpu.make_async_copy(
                w_hbm.at[pl.ds(c * ck, ck), :], wf_ref.at[c], sems.at[c]
            ).start()
        for c in range(_NCHUNK):
            pltpu.make_async_copy(
                wf_ref.at[c], wf_ref.at[c], sems.at[c]
            ).wait()
            w_chunk = wf_ref[c].astype(jnp.bfloat16)
            wb_ref[pl.ds(c * ck, ck), :] = w_chunk
            o_ref[:, pl.ds(c * ck, ck)] = lax.dot_general(
                x_bf,
                w_chunk,
                dimension_numbers=(((1,), (1,)), ((), ())),
                preferred_element_type=jnp.float32,
            )

    @pl.when(pl.program_id(1) != 0)
    def _steady():
        o_ref[...] = lax.dot_general(
            x_bf,
            wb_ref[...],
            dimension_numbers=(((1,), (1,)), ((), ())),
            preferred_element_type=jnp.float32,
        )


@functools.partial(jax.jit, static_argnames=("tm",))
def _solve(xp, m_param, tm=512):
    B, M, N = xp.shape
    K = m_param.shape[0]
    rows = B * M
    x2d = xp.reshape(rows, N)

    tm = min(tm, rows)
    grid_m = pl.cdiv(rows, tm)
    inner = grid_m // 2 if grid_m % 2 == 0 else grid_m
    outer = grid_m // inner

    out = pl.pallas_call(
        _gemm_kernel,
        out_shape=jax.ShapeDtypeStruct((rows, K), jnp.float32),
        grid=(outer, inner),
        in_specs=[
            pl.BlockSpec((tm, N), lambda i, j: (i * inner + j, 0)),
            pl.BlockSpec(memory_space=pl.ANY),
        ],
        out_specs=pl.BlockSpec((tm, K), lambda i, j: (i * inner + j, 0)),
        scratch_shapes=[
            pltpu.VMEM((K, N), jnp.bfloat16),
            pltpu.VMEM((_NCHUNK, K // _NCHUNK, N), jnp.float32),
            pltpu.SemaphoreType.DMA((_NCHUNK,)),
        ],
        compiler_params=pltpu.CompilerParams(
            dimension_semantics=("parallel", "arbitrary"),
            vmem_limit_bytes=56 << 20,
        ),
    )(x2d, m_param)
    return out.reshape(B, M, K)


def kernel(xp, m_param):
    return _solve(xp, m_param)


# speedup vs baseline: 1.0890x; 1.0890x over previous
import functools

import jax
import jax.numpy as jnp
from jax import lax
from jax.experimental import pallas as pl
from jax.experimental.pallas import tpu as pltpu


def _gemm_kernel(x_ref, w_ref, o_ref, wb_ref):
    K = wb_ref.shape[0]
    h = K // 2

    @pl.when(pl.program_id(1) == 0)
    def _cast_weight():
        wb_ref[...] = w_ref[...].astype(jnp.bfloat16)

    x_bf = x_ref[...].astype(jnp.bfloat16)
    o_ref[:, :h] = lax.dot_general(
        x_bf,
        wb_ref[:h, :],
        dimension_numbers=(((1,), (1,)), ((), ())),
        preferred_element_type=jnp.float32,
    )
    o_ref[:, h:] = lax.dot_general(
        x_bf,
        wb_ref[h:, :],
        dimension_numbers=(((1,), (1,)), ((), ())),
        preferred_element_type=jnp.float32,
    )


@functools.partial(jax.jit, static_argnames=("tm",))
def _solve(xp, m_param, tm=512):
    B, M, N = xp.shape
    K = m_param.shape[0]
    rows = B * M
    x2d = xp.reshape(rows, N)

    tm = min(tm, rows)
    grid_m = pl.cdiv(rows, tm)
    inner = grid_m // 2 if grid_m % 2 == 0 else grid_m
    outer = grid_m // inner

    out = pl.pallas_call(
        _gemm_kernel,
        out_shape=jax.ShapeDtypeStruct((rows, K), jnp.float32),
        grid=(outer, inner),
        in_specs=[
            pl.BlockSpec((tm, N), lambda i, j: (i * inner + j, 0)),
            pl.BlockSpec((K, N), lambda i, j: (0, 0)),
        ],
        out_specs=pl.BlockSpec((tm, K), lambda i, j: (i * inner + j, 0)),
        scratch_shapes=[pltpu.VMEM((K, N), jnp.bfloat16)],
        compiler_params=pltpu.CompilerParams(
            dimension_semantics=("parallel", "arbitrary"),
            vmem_limit_bytes=56 << 20,
        ),
    )(x2d, m_param)
    return out.reshape(B, M, K)


def kernel(xp, m_param):
    return _solve(xp, m_param)
